# baseline (speedup 1.0000x reference)
import jax
import jax.numpy as jnp
from jax.experimental import pallas as pl

B, S, D = 4, 4096, 2048
ROWS = B * S
BLK_R = 512


def _select_body(mask_ref, emb_ref, mte_ref, out_ref):
    m = mask_ref[...] != 0
    out_ref[...] = jnp.where(m, mte_ref[...], emb_ref[...])


def kernel(embeddings, token_mask, mask_token_embed):
    emb = embeddings.reshape(ROWS, D)
    mask = token_mask.reshape(ROWS, 1).astype(jnp.int32)
    mte = mask_token_embed.reshape(1, D)

    out = pl.pallas_call(
        _select_body,
        grid=(ROWS // BLK_R,),
        in_specs=[
            pl.BlockSpec((BLK_R, 1), lambda i: (i, 0)),
            pl.BlockSpec((BLK_R, D), lambda i: (i, 0)),
            pl.BlockSpec((1, D), lambda i: (0, 0)),
        ],
        out_specs=pl.BlockSpec((BLK_R, D), lambda i: (i, 0)),
        out_shape=jax.ShapeDtypeStruct((ROWS, D), jnp.float32),
    )(mask, emb, mte)
    return out.reshape(B, S, D)
